# baseline (speedup 1.0000x reference)
import jax
import jax.numpy as jnp
from jax import lax
from jax.experimental import pallas as pl
from jax.experimental.pallas import tpu as pltpu

_B = 1024
_C = 100000

_RB = 256
_CB = 4096
_NCB = (_C + _CB - 1) // _CB


def _body(x_ref, lab_ref, perm_ref, co_ref, out_ref,
          m_ref, s_ref, g_ref):
    r = pl.program_id(0)
    c = pl.program_id(1)
    nr = pl.num_programs(0)
    nc = pl.num_programs(1)

    @pl.when(c == 0)
    def _init():
        m_ref[...] = jnp.full_like(m_ref, -jnp.inf)
        s_ref[...] = jnp.zeros_like(s_ref)
        g_ref[...] = jnp.zeros_like(g_ref)

    xb = x_ref[...]
    col = c * _CB + lax.broadcasted_iota(jnp.int32, (_RB, _CB), 1)

    lab = lab_ref[...]
    perm = perm_ref[...]
    co = co_ref[...]
    wx = jnp.where(col == lab, co * xb, 0.0) \
        + jnp.where(col == perm, (1.0 - co) * xb, 0.0)
    g_ref[...] += jnp.sum(wx, axis=1, keepdims=True)

    def update(xm):
        bm = jnp.max(xm, axis=1, keepdims=True)
        m_old = m_ref[...]
        m_new = jnp.maximum(m_old, bm)
        p = jnp.exp(xm - m_new)
        bs = jnp.sum(p, axis=1, keepdims=True)
        s_new = s_ref[...] * jnp.exp(m_old - m_new) + bs
        m_ref[...] = m_new
        s_ref[...] = s_new
        return m_new, s_new

    @pl.when(c < nc - 1)
    def _main():
        update(xb)

    @pl.when(c == nc - 1)
    def _fin():
        m_new, s_new = update(jnp.where(col < _C, xb, -jnp.inf))
        lse = m_new + jnp.log(s_new)
        part = jnp.sum(lse - g_ref[...])
        prev = jnp.where(r == 0, 0.0, out_ref[0, 0])
        acc = prev + part
        out_ref[0, 0] = jnp.where(r == nr - 1, acc * (1.0 / _B), acc)


@jax.jit
def _loss(x, lab2, perm2, co2):
    return pl.pallas_call(
        _body,
        grid=(_B // _RB, _NCB),
        in_specs=[
            pl.BlockSpec((_RB, _CB), lambda r, c: (r, c)),
            pl.BlockSpec((_RB, 1), lambda r, c: (r, 0)),
            pl.BlockSpec((_RB, 1), lambda r, c: (r, 0)),
            pl.BlockSpec((_RB, 1), lambda r, c: (r, 0)),
        ],
        out_specs=pl.BlockSpec(memory_space=pltpu.SMEM),
        out_shape=jax.ShapeDtypeStruct((1, 1), jnp.float32),
        scratch_shapes=[
            pltpu.VMEM((_RB, 1), jnp.float32),
            pltpu.VMEM((_RB, 1), jnp.float32),
            pltpu.VMEM((_RB, 1), jnp.float32),
        ],
        compiler_params=pltpu.CompilerParams(
            dimension_semantics=("arbitrary", "arbitrary"),
        ),
    )(x, lab2, perm2, co2)


def kernel(x, labels, perm_labels, label_coeffs):
    lab2 = labels.astype(jnp.int32).reshape(_B, 1)
    perm2 = perm_labels.astype(jnp.int32).reshape(_B, 1)
    co2 = label_coeffs.astype(jnp.float32).reshape(_B, 1)
    return _loss(x, lab2, perm2, co2)[0, 0]


# speedup vs baseline: 1.0475x; 1.0475x over previous
import jax
import jax.numpy as jnp
from jax import lax
from jax.experimental import pallas as pl
from jax.experimental.pallas import tpu as pltpu

_B = 1024
_C = 100000

_RB = 256
_CB = 8192
_NCB = (_C + _CB - 1) // _CB


def _body(x_ref, lab_ref, perm_ref, co_ref, out_ref,
          m_ref, s_ref, xl_ref, xp_ref):
    r = pl.program_id(0)
    c = pl.program_id(1)
    nr = pl.num_programs(0)
    nc = pl.num_programs(1)

    @pl.when(c == 0)
    def _init():
        m_ref[...] = jnp.full_like(m_ref, -jnp.inf)
        s_ref[...] = jnp.zeros_like(s_ref)
        xl_ref[...] = jnp.zeros_like(xl_ref)
        xp_ref[...] = jnp.zeros_like(xp_ref)

    xb = x_ref[...]
    col = c * _CB + lax.broadcasted_iota(jnp.int32, (_RB, _CB), 1)

    lab = lab_ref[...]
    perm = perm_ref[...]
    xl_ref[...] += jnp.sum(jnp.where(col == lab, xb, 0.0),
                           axis=1, keepdims=True)
    xp_ref[...] += jnp.sum(jnp.where(col == perm, xb, 0.0),
                           axis=1, keepdims=True)

    def update(xm):
        bm = jnp.max(xm, axis=1, keepdims=True)
        m_old = m_ref[...]
        m_new = jnp.maximum(m_old, bm)
        p = jnp.exp(xm - m_new)
        bs = jnp.sum(p, axis=1, keepdims=True)
        s_new = s_ref[...] * jnp.exp(m_old - m_new) + bs
        m_ref[...] = m_new
        s_ref[...] = s_new
        return m_new, s_new

    @pl.when(c < nc - 1)
    def _main():
        update(xb)

    @pl.when(c == nc - 1)
    def _fin():
        m_new, s_new = update(jnp.where(col < _C, xb, -jnp.inf))
        lse = m_new + jnp.log(s_new)
        co = co_ref[...]
        part = jnp.sum(lse - co * xl_ref[...]
                       - (1.0 - co) * xp_ref[...])
        prev = jnp.where(r == 0, 0.0, out_ref[0, 0])
        acc = prev + part
        out_ref[0, 0] = jnp.where(r == nr - 1, acc * (1.0 / _B), acc)


@jax.jit
def _loss(x, lab2, perm2, co2):
    return pl.pallas_call(
        _body,
        grid=(_B // _RB, _NCB),
        in_specs=[
            pl.BlockSpec((_RB, _CB), lambda r, c: (r, c)),
            pl.BlockSpec((_RB, 1), lambda r, c: (r, 0)),
            pl.BlockSpec((_RB, 1), lambda r, c: (r, 0)),
            pl.BlockSpec((_RB, 1), lambda r, c: (r, 0)),
        ],
        out_specs=pl.BlockSpec(memory_space=pltpu.SMEM),
        out_shape=jax.ShapeDtypeStruct((1, 1), jnp.float32),
        scratch_shapes=[
            pltpu.VMEM((_RB, 1), jnp.float32),
            pltpu.VMEM((_RB, 1), jnp.float32),
            pltpu.VMEM((_RB, 1), jnp.float32),
            pltpu.VMEM((_RB, 1), jnp.float32),
        ],
        compiler_params=pltpu.CompilerParams(
            dimension_semantics=("arbitrary", "arbitrary"),
        ),
    )(x, lab2, perm2, co2)


def kernel(x, labels, perm_labels, label_coeffs):
    lab2 = labels.astype(jnp.int32).reshape(_B, 1)
    perm2 = perm_labels.astype(jnp.int32).reshape(_B, 1)
    co2 = label_coeffs.astype(jnp.float32).reshape(_B, 1)
    return _loss(x, lab2, perm2, co2)[0, 0]


# speedup vs baseline: 1.1180x; 1.0673x over previous
import jax
import jax.numpy as jnp
from jax import lax
from jax.experimental import pallas as pl
from jax.experimental.pallas import tpu as pltpu

_B = 1024
_C = 100000

_RB = 256
_CB = 12544
_NCB = (_C + _CB - 1) // _CB


def _body(x_ref, lab_ref, perm_ref, co_ref, out_ref,
          m_ref, s_ref, xl_ref, xp_ref):
    r = pl.program_id(0)
    c = pl.program_id(1)
    nr = pl.num_programs(0)
    nc = pl.num_programs(1)

    @pl.when(c == 0)
    def _init():
        m_ref[...] = jnp.full_like(m_ref, -jnp.inf)
        s_ref[...] = jnp.zeros_like(s_ref)
        xl_ref[...] = jnp.zeros_like(xl_ref)
        xp_ref[...] = jnp.zeros_like(xp_ref)

    xb = x_ref[...]
    col = c * _CB + lax.broadcasted_iota(jnp.int32, (_RB, _CB), 1)

    lab = lab_ref[...]
    perm = perm_ref[...]
    xl_ref[...] += jnp.sum(jnp.where(col == lab, xb, 0.0),
                           axis=1, keepdims=True)
    xp_ref[...] += jnp.sum(jnp.where(col == perm, xb, 0.0),
                           axis=1, keepdims=True)

    def update(xm):
        bm = jnp.max(xm, axis=1, keepdims=True)
        m_old = m_ref[...]
        m_new = jnp.maximum(m_old, bm)
        p = jnp.exp(xm - m_new)
        bs = jnp.sum(p, axis=1, keepdims=True)
        s_new = s_ref[...] * jnp.exp(m_old - m_new) + bs
        m_ref[...] = m_new
        s_ref[...] = s_new
        return m_new, s_new

    @pl.when(c < nc - 1)
    def _main():
        update(xb)

    @pl.when(c == nc - 1)
    def _fin():
        m_new, s_new = update(jnp.where(col < _C, xb, -jnp.inf))
        lse = m_new + jnp.log(s_new)
        co = co_ref[...]
        part = jnp.sum(lse - co * xl_ref[...]
                       - (1.0 - co) * xp_ref[...])
        prev = jnp.where(r == 0, 0.0, out_ref[0, 0])
        acc = prev + part
        out_ref[0, 0] = jnp.where(r == nr - 1, acc * (1.0 / _B), acc)


@jax.jit
def _loss(x, lab2, perm2, co2):
    return pl.pallas_call(
        _body,
        grid=(_B // _RB, _NCB),
        in_specs=[
            pl.BlockSpec((_RB, _CB), lambda r, c: (r, c)),
            pl.BlockSpec((_RB, 1), lambda r, c: (r, 0)),
            pl.BlockSpec((_RB, 1), lambda r, c: (r, 0)),
            pl.BlockSpec((_RB, 1), lambda r, c: (r, 0)),
        ],
        out_specs=pl.BlockSpec(memory_space=pltpu.SMEM),
        out_shape=jax.ShapeDtypeStruct((1, 1), jnp.float32),
        scratch_shapes=[
            pltpu.VMEM((_RB, 1), jnp.float32),
            pltpu.VMEM((_RB, 1), jnp.float32),
            pltpu.VMEM((_RB, 1), jnp.float32),
            pltpu.VMEM((_RB, 1), jnp.float32),
        ],
        compiler_params=pltpu.CompilerParams(
            dimension_semantics=("arbitrary", "arbitrary"),
        ),
    )(x, lab2, perm2, co2)


def kernel(x, labels, perm_labels, label_coeffs):
    lab2 = labels.astype(jnp.int32).reshape(_B, 1)
    perm2 = perm_labels.astype(jnp.int32).reshape(_B, 1)
    co2 = label_coeffs.astype(jnp.float32).reshape(_B, 1)
    return _loss(x, lab2, perm2, co2)[0, 0]


# speedup vs baseline: 1.1199x; 1.0017x over previous
import jax
import jax.numpy as jnp
from jax import lax
from jax.experimental import pallas as pl
from jax.experimental.pallas import tpu as pltpu

_B = 1024
_C = 100000

_RB = 256
_CB = 6272
_NCB = 16
_HC = _NCB // 2


def _gather_acc(col, xb, lab, perm, xl_ref, xp_ref):
    xl_ref[...] += jnp.sum(jnp.where(col == lab, xb, 0.0),
                           axis=1, keepdims=True)
    xp_ref[...] += jnp.sum(jnp.where(col == perm, xb, 0.0),
                           axis=1, keepdims=True)


def _body(xa_ref, xb_ref, lab_ref, perm_ref, co_ref, out_ref,
          m_ref, s_ref, xl_ref, xp_ref):
    r = pl.program_id(0)
    c = pl.program_id(1)
    nr = pl.num_programs(0)
    nc = pl.num_programs(1)

    @pl.when(c == 0)
    def _init():
        m_ref[...] = jnp.full_like(m_ref, -jnp.inf)
        s_ref[...] = jnp.zeros_like(s_ref)
        xl_ref[...] = jnp.zeros_like(xl_ref)
        xp_ref[...] = jnp.zeros_like(xp_ref)

    xa = xa_ref[...]
    xb = xb_ref[...]
    iota = lax.broadcasted_iota(jnp.int32, (_RB, _CB), 1)
    col_a = c * _CB + iota
    col_b = (_HC + c) * _CB + iota

    lab = lab_ref[...]
    perm = perm_ref[...]
    _gather_acc(col_a, xa, lab, perm, xl_ref, xp_ref)
    _gather_acc(col_b, xb, lab, perm, xl_ref, xp_ref)

    def update(xm):
        bm = jnp.max(xm, axis=1, keepdims=True)
        m_old = m_ref[...]
        m_new = jnp.maximum(m_old, bm)
        p = jnp.exp(xm - m_new)
        bs = jnp.sum(p, axis=1, keepdims=True)
        s_new = s_ref[...] * jnp.exp(m_old - m_new) + bs
        m_ref[...] = m_new
        s_ref[...] = s_new
        return m_new, s_new

    update(xa)

    @pl.when(c < nc - 1)
    def _main():
        update(xb)

    @pl.when(c == nc - 1)
    def _fin():
        m_new, s_new = update(jnp.where(col_b < _C, xb, -jnp.inf))
        lse = m_new + jnp.log(s_new)
        co = co_ref[...]
        part = jnp.sum(lse - co * xl_ref[...]
                       - (1.0 - co) * xp_ref[...])
        prev = jnp.where(r == 0, 0.0, out_ref[0, 0])
        acc = prev + part
        out_ref[0, 0] = jnp.where(r == nr - 1, acc * (1.0 / _B), acc)


@jax.jit
def _loss(x, lab2, perm2, co2):
    return pl.pallas_call(
        _body,
        grid=(_B // _RB, _HC),
        in_specs=[
            pl.BlockSpec((_RB, _CB), lambda r, c: (r, c)),
            pl.BlockSpec((_RB, _CB), lambda r, c: (r, _HC + c)),
            pl.BlockSpec((_RB, 1), lambda r, c: (r, 0)),
            pl.BlockSpec((_RB, 1), lambda r, c: (r, 0)),
            pl.BlockSpec((_RB, 1), lambda r, c: (r, 0)),
        ],
        out_specs=pl.BlockSpec(memory_space=pltpu.SMEM),
        out_shape=jax.ShapeDtypeStruct((1, 1), jnp.float32),
        scratch_shapes=[
            pltpu.VMEM((_RB, 1), jnp.float32),
            pltpu.VMEM((_RB, 1), jnp.float32),
            pltpu.VMEM((_RB, 1), jnp.float32),
            pltpu.VMEM((_RB, 1), jnp.float32),
        ],
        compiler_params=pltpu.CompilerParams(
            dimension_semantics=("arbitrary", "arbitrary"),
        ),
    )(x, x, lab2, perm2, co2)


def kernel(x, labels, perm_labels, label_coeffs):
    lab2 = labels.astype(jnp.int32).reshape(_B, 1)
    perm2 = perm_labels.astype(jnp.int32).reshape(_B, 1)
    co2 = label_coeffs.astype(jnp.float32).reshape(_B, 1)
    return _loss(x, lab2, perm2, co2)[0, 0]
